# baseline (speedup 1.0000x reference)
import functools

import jax
import jax.numpy as jnp
from jax import lax
from jax.experimental import pallas as pl
from jax.experimental.pallas import tpu as pltpu
from jax.experimental.pallas import tpu_sc as plsc

BLK = 128
NBUF = 4
AHEAD = 3
NTRS = 2


def _make_gather(V, D, N):
    info = plsc.get_sparse_core_info()
    NC, NS = info.num_cores, info.num_subcores
    NW = NC * NS
    assert N % (NW * BLK * NBUF) == 0
    b_per_w = N // NW
    n_blk = b_per_w // BLK
    n_groups = n_blk // NBUF
    n_dtile = D // 8
    n_tiles = (N * D) // (8 * 128)
    mesh = plsc.VectorSubcoreMesh(core_axis_name="c", subcore_axis_name="s")

    @functools.partial(
        pl.kernel,
        mesh=mesh,
        out_type=jax.ShapeDtypeStruct((n_tiles, 8, 128), jnp.float32),
        scratch_types=[
            pltpu.VMEM((b_per_w,), jnp.int32),
            pltpu.VMEM((NBUF, BLK, D), jnp.float32),
            pltpu.VMEM((NTRS, n_dtile, 8, 128), jnp.float32),
            [pltpu.SemaphoreType.DMA] * NBUF,
            [pltpu.SemaphoreType.DMA] * NTRS,
        ],
        compiler_params=pltpu.CompilerParams(
            use_tc_tiling_on_sc=False, needs_layout_passes=False
        ),
    )
    def gather_kernel(table_hbm, idx_hbm, out_hbm, idx_all, rows_v, trs_v,
                      sem_g, sem_o):
        wid = lax.axis_index("s") * NC + lax.axis_index("c")
        base = wid * b_per_w
        pltpu.sync_copy(idx_hbm.at[pl.ds(base, b_per_w)], idx_all)

        iota16 = lax.iota(jnp.int32, 16)
        row_vecs = [iota16 + (g * 16) for g in range(8)]

        def fire(c, s):
            pltpu.async_copy(
                table_hbm.at[idx_all.at[pl.ds(c * BLK, BLK)]],
                rows_v.at[s], sem_g[s],
            )

        def drain_gather(s):
            pltpu.make_async_copy(
                table_hbm.at[pl.ds(0, BLK)], rows_v.at[s], sem_g[s]
            ).wait()

        def transpose(s, t):
            rows = rows_v.at[s]
            trs = trs_v.at[t]

            def dbody(d, carry):
                dcol = jnp.full((16,), d, jnp.int32)
                dt = d >> 3
                dr = d & 7
                for g in range(8):
                    v = plsc.load_gather(rows, [row_vecs[g], dcol])
                    trs[dt, dr, pl.ds(g * 16, 16)] = v
                return carry

            lax.fori_loop(0, D, dbody, 0)

        def fire_out(c, t):
            m = wid * n_blk + c
            f = m >> 7
            bt = m & 127
            for dt in range(n_dtile):
                pltpu.async_copy(
                    trs_v.at[t].at[dt],
                    out_hbm.at[(f * n_dtile + dt) * 128 + bt],
                    sem_o[t],
                )

        def wait_out(t):
            pltpu.make_async_copy(
                trs_v.at[t], out_hbm.at[pl.ds(0, n_dtile)], sem_o[t]
            ).wait()

        def block(c, s, t, do_wait_out, do_fire):
            if do_wait_out:
                wait_out(t)
            if do_fire:
                fire(c + AHEAD, (s + AHEAD) % NBUF)
            drain_gather(s)
            transpose(s, t)
            fire_out(c, t)

        for c0 in range(AHEAD):
            fire(c0, c0)
        for b in range(NBUF):
            block(b, b, b % NTRS, b >= NTRS, True)

        def body(g, carry):
            for b in range(NBUF):
                c = g * NBUF + b
                block(c, b, b % NTRS, True, True)
            return carry

        lax.fori_loop(1, n_groups - 1, body, 0)

        for b in range(NBUF):
            c = (n_groups - 1) * NBUF + b
            block(c, b, b % NTRS, True, b == 0)
        for t in range(NTRS):
            wait_out(t)

    return gather_kernel


def kernel(data, table):
    B, F = data.shape
    V, D = table.shape
    idx = data.T.reshape(-1).astype(jnp.int32)
    tiles = _make_gather(V, D, B * F)(table, idx)
    out5 = tiles.reshape(F, D // 8, B // 128, 8, 128)
    return out5.transpose(2, 4, 0, 1, 3).reshape(B, F, D)
